# Initial kernel scaffold; baseline (speedup 1.0000x reference)
import jax
import jax.numpy as jnp
from jax.experimental import pallas as pl


def kernel(x, edge_index, batch, W_enc, b_enc, Wl, bl, Wr, gamma, beta, W_h1, b_h1, g_h, be_h, W_h2, b_h2):
    raise NotImplementedError("write your pallas kernel here")


# speedup vs baseline: 5.9002x; 5.9002x over previous
import functools

import jax
import jax.numpy as jnp
from jax import lax
from jax.experimental import pallas as pl
from jax.experimental.pallas import tpu as pltpu
import jax.experimental.pallas.tpu_sc as plsc

N = 10000
E = 320000
DIN = 128
H = 256
HH = 128
DEMB = 128
G = 16
L = 3

NSUB = 16
CK = 125
CH = E // NSUB // CK
CH_DEG = E // 2 // NSUB // CK
NP = 10240
TROWS = NP // NSUB
WB = TROWS // 128
IDXB = 32
NIB = CH // IDXB

BLK = 1000
GRID = N // BLK

def _sc_mesh():
    return plsc.VectorSubcoreMesh(core_axis_name="c", subcore_axis_name="s",
                                  num_cores=2, num_subcores=NSUB)


def _zero_vmem_rows(buf, rows, width):
    z = jnp.zeros((16,), jnp.float32)

    def zrow(i, carry):
        for j in range(width // 16):
            buf[i, pl.ds(j * 16, 16)] = z
        return carry

    lax.fori_loop(0, rows, zrow, 0)


@functools.lru_cache(maxsize=None)
def _sc_deg_kernel():
    return functools.partial(
        pl.kernel,
        out_type=jax.ShapeDtypeStruct((2, NP, 16), jnp.float32),
        mesh=_sc_mesh(),
        scratch_types=[
            pltpu.VMEM((CH_DEG, CK), jnp.int32),
            pltpu.VMEM((128, 16), jnp.float32),
            pltpu.VMEM((CK, 16), jnp.float32),
            pltpu.VMEM_SHARED((NP, 16), jnp.float32),
        ],
    )(_sc_deg_body)


def _sc_deg(dst):
    return _sc_deg_kernel()(dst)


def _sc_deg_body(dst_hbm, out_hbm, dst_v, zbuf_v, ones_v, deg_sh):
    c = lax.axis_index("c")
    s = lax.axis_index("s")
    _zero_vmem_rows(zbuf_v, 128, 16)
    for k in range(WB):
        pltpu.sync_copy(zbuf_v, deg_sh.at[pl.ds(s * TROWS + k * 128, 128)])
    plsc.subcore_barrier()
    one = jnp.ones((16,), jnp.float32)

    def orow(i, carry):
        ones_v[i, pl.ds(0, 16)] = one
        return carry

    lax.fori_loop(0, CK, orow, 0)
    pltpu.sync_copy(dst_hbm.at[c, s], dst_v)

    def body(j, carry):
        pltpu.sync_copy(ones_v, deg_sh.at[dst_v.at[j]], add=True)
        return carry

    lax.fori_loop(0, CH_DEG, body, 0)
    plsc.subcore_barrier()
    for k in range(WB):
        sl = pl.ds(s * TROWS + k * 128, 128)
        pltpu.sync_copy(deg_sh.at[sl], out_hbm.at[c, sl])


@functools.lru_cache(maxsize=None)
def _sc_agg_kernel():
    return functools.partial(
        pl.kernel,
        out_type=jax.ShapeDtypeStruct((2, NP, HH), jnp.float32),
        mesh=_sc_mesh(),
        scratch_types=[
            pltpu.VMEM((IDXB, CK), jnp.int32),
            pltpu.VMEM((IDXB, CK), jnp.int32),
            pltpu.VMEM((64, HH), jnp.float32),
            pltpu.VMEM((CK, HH), jnp.float32),
            pltpu.VMEM_SHARED((NP, HH), jnp.float32),
            pltpu.SemaphoreType.DMA,
        ],
    )(_sc_agg_body)


def _sc_agg(h_flat, src_idx, dst3):
    return _sc_agg_kernel()(h_flat, src_idx, dst3)


def _sc_agg_body(h_hbm, src_hbm, dst_hbm, out_hbm, src_v, dst_v, zbuf_v,
                 buf_v, agg_sh, sem):
    c = lax.axis_index("c")
    s = lax.axis_index("s")
    _zero_vmem_rows(zbuf_v, 64, HH)
    for k in range(TROWS // 64):
        pltpu.sync_copy(zbuf_v, agg_sh.at[pl.ds(s * TROWS + k * 64, 64)])
    plsc.subcore_barrier()

    def body(j, carry):
        pltpu.async_copy(h_hbm.at[src_v.at[j]], buf_v, sem).wait()
        pltpu.sync_copy(buf_v, agg_sh.at[dst_v.at[j]], add=True)
        return carry

    for b in range(NIB):
        pltpu.sync_copy(src_hbm.at[c, s, pl.ds(b * IDXB, IDXB)], src_v)
        pltpu.sync_copy(dst_hbm.at[s, pl.ds(b * IDXB, IDXB)], dst_v)
        lax.fori_loop(0, IDXB, body, 0)
    plsc.subcore_barrier()
    for k in range(WB):
        sl = pl.ds(s * TROWS + k * 128, 128)
        pltpu.sync_copy(agg_sh.at[sl], out_hbm.at[c, sl])


def _dot(a, b):
    return jax.lax.dot(a, b, preferred_element_type=jnp.float32)


def _enc_body(x_ref, w_ref, b_ref, out_ref):
    h = _dot(x_ref[...], w_ref[...]) + b_ref[...]
    out_ref[0] = h[:, :HH]
    out_ref[1] = h[:, HH:]


def _enc(x, w, b):
    return pl.pallas_call(
        _enc_body,
        grid=(GRID,),
        in_specs=[
            pl.BlockSpec((BLK, DIN), lambda i: (i, 0)),
            pl.BlockSpec((DIN, H), lambda i: (0, 0)),
            pl.BlockSpec((1, H), lambda i: (0, 0)),
        ],
        out_specs=pl.BlockSpec((2, BLK, HH), lambda i: (0, i, 0)),
        out_shape=jax.ShapeDtypeStruct((2, N, HH), jnp.float32),
    )(x, w, b)


def _passA_body(agg_ref, degi_ref, h_ref, wl_ref, bl_ref, wr_ref,
                t_ref, stats_ref, acc):
    i = pl.program_id(0)
    degi = degi_ref[...]
    a0 = agg_ref[0] * degi
    a1 = agg_ref[1] * degi
    h0 = h_ref[0]
    h1 = h_ref[1]
    u = (_dot(a0, wl_ref[:HH, :]) + _dot(a1, wl_ref[HH:, :])
         + _dot(h0, wr_ref[:HH, :]) + _dot(h1, wr_ref[HH:, :])
         + bl_ref[...])
    t = jnp.maximum(u, 0.0)
    t0 = t[:, :HH] + h0
    t1 = t[:, HH:] + h1
    t_ref[0] = t0
    t_ref[1] = t1
    tfull = jnp.concatenate([t0, t1], axis=1)

    @pl.when(i == 0)
    def _():
        acc[...] = jnp.zeros_like(acc)

    acc[0:1, :] += jnp.sum(tfull, axis=0, keepdims=True)
    acc[1:2, :] += jnp.sum(tfull * tfull, axis=0, keepdims=True)

    @pl.when(i == GRID - 1)
    def _():
        stats_ref[...] = acc[...]


def _passA(agg, degi, h, wl, bl, wr):
    return pl.pallas_call(
        _passA_body,
        grid=(GRID,),
        in_specs=[
            pl.BlockSpec((2, BLK, HH), lambda i: (0, i, 0)),
            pl.BlockSpec((BLK, 1), lambda i: (i, 0)),
            pl.BlockSpec((2, BLK, HH), lambda i: (0, i, 0)),
            pl.BlockSpec((H, H), lambda i: (0, 0)),
            pl.BlockSpec((1, H), lambda i: (0, 0)),
            pl.BlockSpec((H, H), lambda i: (0, 0)),
        ],
        out_specs=[
            pl.BlockSpec((2, BLK, HH), lambda i: (0, i, 0)),
            pl.BlockSpec((8, H), lambda i: (0, 0)),
        ],
        out_shape=[
            jax.ShapeDtypeStruct((2, N, HH), jnp.float32),
            jax.ShapeDtypeStruct((8, H), jnp.float32),
        ],
        scratch_shapes=[pltpu.VMEM((8, H), jnp.float32)],
    )(agg, degi, h, wl, bl, wr)


def _passB_body(t_ref, stats_ref, gamma_ref, beta_ref, batch_ref,
                h_ref, pmax_ref, psum_ref, cnt_ref,
                macc, sacc, cacc):
    i = pl.program_id(0)
    mu = stats_ref[0:1, :] * (1.0 / N)
    var = stats_ref[1:2, :] * (1.0 / N) - mu * mu
    scale = lax.rsqrt(var + 1e-5) * gamma_ref[...]
    shift = beta_ref[...] - mu * scale
    tfull = jnp.concatenate([t_ref[0], t_ref[1]], axis=1)
    hn = tfull * scale + shift
    h_ref[0] = hn[:, :HH]
    h_ref[1] = hn[:, HH:]

    @pl.when(i == 0)
    def _():
        macc[...] = jnp.full_like(macc, -jnp.inf)
        sacc[...] = jnp.zeros_like(sacc)
        cacc[...] = jnp.zeros_like(cacc)

    b = batch_ref[...]
    bmin = jnp.min(b)
    bmax = jnp.max(b)
    for g in range(G):
        @pl.when((g >= bmin) & (g <= bmax))
        def _():
            sel = b == g
            masked_max = jnp.max(jnp.where(sel, hn, -jnp.inf), axis=0,
                                 keepdims=True)
            masked_sum = jnp.sum(jnp.where(sel, hn, 0.0), axis=0,
                                 keepdims=True)
            nsel = jnp.sum(sel.astype(jnp.float32))
            macc[g:g + 1, :] = jnp.maximum(macc[g:g + 1, :], masked_max)
            sacc[g:g + 1, :] += masked_sum
            cacc[g:g + 1, :] += nsel

    @pl.when(i == GRID - 1)
    def _():
        pmax_ref[...] = macc[...]
        psum_ref[...] = sacc[...]
        cnt_ref[...] = cacc[...]


def _passB(t, stats, gamma, beta, batchcol):
    return pl.pallas_call(
        _passB_body,
        grid=(GRID,),
        in_specs=[
            pl.BlockSpec((2, BLK, HH), lambda i: (0, i, 0)),
            pl.BlockSpec((8, H), lambda i: (0, 0)),
            pl.BlockSpec((1, H), lambda i: (0, 0)),
            pl.BlockSpec((1, H), lambda i: (0, 0)),
            pl.BlockSpec((BLK, 1), lambda i: (i, 0)),
        ],
        out_specs=[
            pl.BlockSpec((2, BLK, HH), lambda i: (0, i, 0)),
            pl.BlockSpec((G, H), lambda i: (0, 0)),
            pl.BlockSpec((G, H), lambda i: (0, 0)),
            pl.BlockSpec((G, 128), lambda i: (0, 0)),
        ],
        out_shape=[
            jax.ShapeDtypeStruct((2, N, HH), jnp.float32),
            jax.ShapeDtypeStruct((G, H), jnp.float32),
            jax.ShapeDtypeStruct((G, H), jnp.float32),
            jax.ShapeDtypeStruct((G, 128), jnp.float32),
        ],
        scratch_shapes=[
            pltpu.VMEM((G, H), jnp.float32),
            pltpu.VMEM((G, H), jnp.float32),
            pltpu.VMEM((G, 128), jnp.float32),
        ],
    )(t, stats, gamma, beta, batchcol)


def _head_body(pmax_ref, psum_ref, cnt_ref, w1_ref, b1_ref, gh_ref,
               beh_ref, w2_ref, b2_ref, out_ref):
    cnt = jnp.maximum(cnt_ref[:, 0:1], 1.0)
    hs_max = jnp.zeros((G, H), jnp.float32)
    hs_sum = jnp.zeros((G, H), jnp.float32)
    for l in range(L):
        pm = pmax_ref[l]
        hs_max = hs_max + jnp.where(jnp.isfinite(pm), pm, 0.0)
        hs_sum = hs_sum + psum_ref[l]
    hs_mean = hs_sum / cnt
    z = _dot(hs_max, w1_ref[:H, :]) + _dot(hs_mean, w1_ref[H:, :]) + b1_ref[...]
    mu = jnp.mean(z, axis=0, keepdims=True)
    var = jnp.mean(z * z, axis=0, keepdims=True) - mu * mu
    zn = (z - mu) * lax.rsqrt(var + 1e-5) * gh_ref[...] + beh_ref[...]
    z = jnp.maximum(zn, 0.0)
    out_ref[...] = _dot(z, w2_ref[...]) + b2_ref[...]


def _head(pmax, psum, cnt, w1, b1, gh, beh, w2, b2):
    return pl.pallas_call(
        _head_body,
        grid=(1,),
        in_specs=[
            pl.BlockSpec((L, G, H), lambda i: (0, 0, 0)),
            pl.BlockSpec((L, G, H), lambda i: (0, 0, 0)),
            pl.BlockSpec((G, 128), lambda i: (0, 0)),
            pl.BlockSpec((2 * H, H), lambda i: (0, 0)),
            pl.BlockSpec((1, H), lambda i: (0, 0)),
            pl.BlockSpec((1, H), lambda i: (0, 0)),
            pl.BlockSpec((1, H), lambda i: (0, 0)),
            pl.BlockSpec((H, DEMB), lambda i: (0, 0)),
            pl.BlockSpec((1, DEMB), lambda i: (0, 0)),
        ],
        out_specs=pl.BlockSpec((G, DEMB), lambda i: (0, 0)),
        out_shape=jax.ShapeDtypeStruct((G, DEMB), jnp.float32),
    )(pmax, psum, cnt, w1, b1, gh, beh, w2, b2)


def kernel(x, edge_index, batch, W_enc, b_enc, Wl, bl, Wr, gamma, beta,
           W_h1, b_h1, g_h, be_h, W_h2, b_h2):
    src = edge_index[0]
    dst = edge_index[1]
    src3 = src.reshape(NSUB, CH, CK)
    src_idx = jnp.stack([src3, src3 + N])
    dst3 = dst.reshape(NSUB, CH, CK)
    dst_deg = dst.reshape(2, NSUB, CH_DEG, CK)
    batchcol = batch.reshape(N, 1)

    deg2 = _sc_deg(dst_deg)
    deg = deg2[0, :N, 0] + deg2[1, :N, 0]
    degi = (1.0 / jnp.maximum(deg, 1.0)).reshape(N, 1)

    h = _enc(x, W_enc, b_enc.reshape(1, H))

    pmaxs = []
    psums = []
    cnt = None
    for l in range(L):
        agg = _sc_agg(h.reshape(2 * N, HH), src_idx, dst3)
        t, stats = _passA(agg, degi, h, Wl[l], bl[l].reshape(1, H), Wr[l])
        h, pmax, psum, cnt = _passB(t, stats, gamma[l].reshape(1, H),
                                    beta[l].reshape(1, H), batchcol)
        pmaxs.append(pmax)
        psums.append(psum)

    out = _head(jnp.stack(pmaxs), jnp.stack(psums), cnt,
                W_h1, b_h1.reshape(1, H), g_h.reshape(1, H),
                be_h.reshape(1, H), W_h2, b_h2.reshape(1, DEMB))
    return out
